# baseline (speedup 1.0000x reference)
import functools

import jax
import jax.numpy as jnp
from jax import lax
from jax.experimental import pallas as pl
from jax.experimental.pallas import tpu as pltpu
from jax.experimental.pallas import tpu_sc as plsc

_B, _N = 4, 131072
_C, _D, _H, _W = 64, 64, 64, 64
_DHW = _D * _H * _W
_NC, _NS, _L = 2, 16, 16
_NWORK = _NC * _NS
_PPW = (_B * _N) // _NWORK
_P = 64
_NCHUNK = _PPW // _P
_NPAIR = _NCHUNK // 2

_INV_SCALE = float(1.0 / (1.0 + 0.1 + 1e-5))
_CLIP_HI = float(1.0 - 1e-5)


_CORNER_OFF = (0, 1, _W, _W + 1, _H * _W, _H * _W + 1, _H * _W + _W, _H * _W + _W + 1)


def _sc_grid_sample(points3, table):
    mesh = plsc.VectorSubcoreMesh(core_axis_name="c", subcore_axis_name="s")

    scratch = (
        [pltpu.VMEM((6 * _P,), jnp.float32)]
        + [pltpu.VMEM((8, _P), jnp.float32) for _ in range(2)]
        + [pltpu.VMEM((_P,), jnp.int32) for _ in range(16)]
        + [pltpu.VMEM((_P, _C), jnp.bfloat16) for _ in range(16)]
        + [pltpu.VMEM((_P, _C), jnp.float32) for _ in range(2)]
        + [pltpu.SemaphoreType.DMA for _ in range(4)]
    )

    @functools.partial(
        pl.kernel,
        out_type=jax.ShapeDtypeStruct((_B * _N, _C), jnp.float32),
        mesh=mesh,
        scratch_types=scratch,
        compiler_params=pltpu.CompilerParams(
            use_tc_tiling_on_sc=False, needs_layout_passes=False),
    )
    def k(points_hbm, table_hbm, out_hbm, *refs):
        ptsv = refs[0]
        w8s = refs[1:3]
        idxs = (refs[3:11], refs[11:19])
        rows = (refs[19:27], refs[27:35])
        ovs = refs[35:37]
        gsems = refs[37:39]
        osems = refs[39:41]

        wid = lax.axis_index("s") * _NC + lax.axis_index("c")
        pbase = wid * _PPW
        tab_off = (pbase // _N) * _DHW

        def load_pair_pts(q):
            off = (wid * (_NCHUNK // 2) + q) * (6 * _P)
            pltpu.sync_copy(points_hbm.at[pl.ds(off, 6 * _P)], ptsv)

        def prep_fire(ci, slot, half):
            w8 = w8s[slot]
            h0 = half * _P
            for g in range(_P // _L):
                sx = pl.ds(h0 + g * _L, _L)
                sy = pl.ds(2 * _P + h0 + g * _L, _L)
                sz = pl.ds(4 * _P + h0 + g * _L, _L)
                s = pl.ds(g * _L, _L)
                fx = jnp.minimum(jnp.maximum(
                    ptsv[sx] * _INV_SCALE + 0.5, 0.0), _CLIP_HI) * (_W - 1.0)
                fy = jnp.minimum(jnp.maximum(
                    ptsv[sy] * _INV_SCALE + 0.5, 0.0), _CLIP_HI) * (_H - 1.0)
                fz = jnp.minimum(jnp.maximum(
                    ptsv[sz] * _INV_SCALE + 0.5, 0.0), _CLIP_HI) * (_D - 1.0)
                x0 = fx.astype(jnp.int32)
                y0 = fy.astype(jnp.int32)
                z0 = fz.astype(jnp.int32)
                wx = fx - x0.astype(jnp.float32)
                wy = fy - y0.astype(jnp.float32)
                wz = fz - z0.astype(jnp.float32)
                ex = 1.0 - wx
                ey = 1.0 - wy
                ez = 1.0 - wz
                base = (z0 * _H + y0) * _W + x0 + tab_off
                for c in range(8):
                    idxs[slot][c][s] = base + _CORNER_OFF[c]
                pzy0 = ez * ey
                pzy1 = ez * wy
                pzy2 = wz * ey
                pzy3 = wz * wy
                w8[0, s] = pzy0 * ex
                w8[1, s] = pzy0 * wx
                w8[2, s] = pzy1 * ex
                w8[3, s] = pzy1 * wx
                w8[4, s] = pzy2 * ex
                w8[5, s] = pzy2 * wx
                w8[6, s] = pzy3 * ex
                w8[7, s] = pzy3 * wx
            for c in range(8):
                pltpu.async_copy(table_hbm.at[idxs[slot][c]], rows[slot][c], gsems[slot])

        def drain_gathers(slot):
            for c in range(8):
                pltpu.make_async_copy(
                    table_hbm.at[idxs[slot][c]], rows[slot][c], gsems[slot]).wait()

        def drain_out(slot):
            pltpu.make_async_copy(
                ovs[slot], out_hbm.at[pl.ds(0, _P)], osems[slot]).wait()

        def accum_store(ci, slot, first):
            n0 = pbase + ci * _P
            w8 = w8s[slot]
            r0, r1, r2, r3, r4, r5, r6, r7 = rows[slot]
            ov = ovs[slot]
            if not first:
                drain_out(slot)

            rr = (r0, r1, r2, r3, r4, r5, r6, r7)

            def grp_body(g, acc_carry):
                p0 = g * _L
                gs = pl.ds(p0, _L)
                wv = [w8[c, gs] for c in range(8)]
                for j in range(_L):
                    p = p0 + j
                    ws = [wv[c][j] for c in range(8)]
                    for kq in range(_C // 32):
                        sl = pl.ds(kq * 32, 32)
                        acc_a = None
                        acc_b = None
                        for c in range(8):
                            a, b = plsc.unpack(
                                rr[c][p, sl], format=plsc.PackFormat.INTERLEAVED,
                                preferred_element_type=jnp.float32)
                            if acc_a is None:
                                acc_a = ws[c] * a
                                acc_b = ws[c] * b
                            else:
                                acc_a = acc_a + ws[c] * a
                                acc_b = acc_b + ws[c] * b
                        ov[p, pl.ds(kq * 32, _L)] = acc_a
                        ov[p, pl.ds(kq * 32 + _L, _L)] = acc_b
                return acc_carry

            lax.fori_loop(0, _P // _L, grp_body, 0)
            pltpu.async_copy(ov, out_hbm.at[pl.ds(n0, _P)], osems[slot])

        load_pair_pts(0)
        prep_fire(0, 0, 0)

        def pair_body(q, carry):
            c0 = 2 * q
            prep_fire(c0 + 1, 1, 1)
            drain_gathers(0)

            @pl.when(q > 0)
            def _():
                drain_out(0)
            accum_store(c0, 0, first=True)

            @pl.when(q < _NPAIR - 1)
            def _():
                load_pair_pts(q + 1)
                prep_fire(c0 + 2, 0, 0)
            drain_gathers(1)

            @pl.when(q > 0)
            def _():
                drain_out(1)
            accum_store(c0 + 1, 1, first=True)
            return carry

        lax.fori_loop(0, _NPAIR, pair_body, 0)
        drain_out(0)
        drain_out(1)

    return k(points3, table)


_CVT = 32768
_TOT = _B * _DHW * _C
_CVT_PW = _TOT // _NWORK
_CVT_NCH = _CVT_PW // _CVT
_CVT_NPAIR = _CVT_NCH // 2


def _sc_convert(table_f32):
    mesh = plsc.VectorSubcoreMesh(core_axis_name="c", subcore_axis_name="s")
    scratch = (
        [pltpu.VMEM((_CVT,), jnp.float32) for _ in range(2)]
        + [pltpu.VMEM((_CVT,), jnp.bfloat16) for _ in range(2)]
        + [pltpu.SemaphoreType.DMA for _ in range(4)]
    )

    @functools.partial(
        pl.kernel,
        out_type=jax.ShapeDtypeStruct((_TOT,), jnp.bfloat16),
        mesh=mesh,
        scratch_types=scratch,
        compiler_params=pltpu.CompilerParams(
            use_tc_tiling_on_sc=False, needs_layout_passes=False),
    )
    def k(src_hbm, dst_hbm, in0, in1, ob0, ob1, is0, is1, os0, os1):
        ins = (in0, in1)
        obs = (ob0, ob1)
        isems = (is0, is1)
        osems = (os0, os1)
        wid = lax.axis_index("s") * _NC + lax.axis_index("c")
        base = wid * _CVT_PW

        def fire_in(ci, slot):
            pltpu.async_copy(src_hbm.at[pl.ds(base + ci * _CVT, _CVT)],
                             ins[slot], isems[slot])

        def drain_in(slot):
            pltpu.make_async_copy(src_hbm.at[pl.ds(0, _CVT)],
                                  ins[slot], isems[slot]).wait()

        def drain_out(slot):
            pltpu.make_async_copy(obs[slot], dst_hbm.at[pl.ds(0, _CVT)],
                                  osems[slot]).wait()

        def convert(ci, slot):
            ib = ins[slot]
            ob = obs[slot]

            def cbody(i, carry):
                o = i * 128
                for u in range(4):
                    a = ib[pl.ds(o + 32 * u, _L)]
                    b = ib[pl.ds(o + 32 * u + _L, _L)]
                    ob[pl.ds(o + 32 * u, 32)] = plsc.pack(
                        a, b, format=plsc.PackFormat.INTERLEAVED)
                return carry

            lax.fori_loop(0, _CVT // 128, cbody, 0)
            pltpu.async_copy(ob, dst_hbm.at[pl.ds(base + ci * _CVT, _CVT)],
                             osems[slot])

        fire_in(0, 0)

        def pair_body(q, carry):
            c0 = 2 * q
            fire_in(c0 + 1, 1)
            drain_in(0)

            @pl.when(q > 0)
            def _():
                drain_out(0)
            convert(c0, 0)

            @pl.when(q < _CVT_NPAIR - 1)
            def _():
                fire_in(c0 + 2, 0)
            drain_in(1)

            @pl.when(q > 0)
            def _():
                drain_out(1)
            convert(c0 + 1, 1)
            return carry

        lax.fori_loop(0, _CVT_NPAIR, pair_body, 0)
        drain_out(0)
        drain_out(1)

    return k(table_f32)


def kernel(points, features_grid):
    B, N, _ = points.shape
    _, C, D, H, W = features_grid.shape
    table_f32 = jnp.transpose(features_grid, (0, 2, 3, 4, 1)).reshape(
        B * D * H * W * C)
    table = _sc_convert(table_f32).reshape(B * D * H * W, C)
    npair = B * N // (2 * _P)
    points3 = jnp.transpose(
        jnp.transpose(points, (2, 0, 1)).reshape(3, npair, 2 * _P),
        (1, 0, 2)).reshape(3 * B * N)
    out = _sc_grid_sample(points3, table)
    return out.reshape(B, N, C)


# speedup vs baseline: 1.4134x; 1.4134x over previous
import functools

import jax
import jax.numpy as jnp
from jax import lax
from jax.experimental import pallas as pl
from jax.experimental.pallas import tpu as pltpu
from jax.experimental.pallas import tpu_sc as plsc

_B, _N = 4, 131072
_C, _D, _H, _W = 64, 64, 64, 64
_DHW = _D * _H * _W
_NC, _NS, _L = 2, 16, 16
_NWORK = _NC * _NS
_PPW = (_B * _N) // _NWORK
_P = 64
_NCHUNK = _PPW // _P
_NPAIR = _NCHUNK // 2

_INV_SCALE = float(1.0 / (1.0 + 0.1 + 1e-5))
_CLIP_HI = float(1.0 - 1e-5)


_CORNER_OFF = (0, 1, _W, _W + 1, _H * _W, _H * _W + 1, _H * _W + _W, _H * _W + _W + 1)


def _sc_grid_sample(points3, table):
    mesh = plsc.VectorSubcoreMesh(core_axis_name="c", subcore_axis_name="s")

    scratch = (
        [pltpu.VMEM((6 * _P,), jnp.float32)]
        + [pltpu.VMEM((8, _P), jnp.float32) for _ in range(2)]
        + [pltpu.VMEM((_P,), jnp.int32) for _ in range(16)]
        + [pltpu.VMEM((_P, _C), jnp.float32) for _ in range(16)]
        + [pltpu.VMEM((_P, _C), jnp.float32) for _ in range(2)]
        + [pltpu.SemaphoreType.DMA for _ in range(4)]
    )

    @functools.partial(
        pl.kernel,
        out_type=jax.ShapeDtypeStruct((_B * _N, _C), jnp.float32),
        mesh=mesh,
        scratch_types=scratch,
        compiler_params=pltpu.CompilerParams(
            use_tc_tiling_on_sc=False, needs_layout_passes=False),
    )
    def k(points_hbm, table_hbm, out_hbm, *refs):
        ptsv = refs[0]
        w8s = refs[1:3]
        idxs = (refs[3:11], refs[11:19])
        rows = (refs[19:27], refs[27:35])
        ovs = refs[35:37]
        gsems = refs[37:39]
        osems = refs[39:41]

        wid = lax.axis_index("s") * _NC + lax.axis_index("c")
        pbase = wid * _PPW
        tab_off = (pbase // _N) * _DHW

        def load_pair_pts(q):
            off = (wid * (_NCHUNK // 2) + q) * (6 * _P)
            pltpu.sync_copy(points_hbm.at[pl.ds(off, 6 * _P)], ptsv)

        def prep_fire(ci, slot, half):
            w8 = w8s[slot]
            h0 = half * _P
            for g in range(_P // _L):
                sx = pl.ds(h0 + g * _L, _L)
                sy = pl.ds(2 * _P + h0 + g * _L, _L)
                sz = pl.ds(4 * _P + h0 + g * _L, _L)
                s = pl.ds(g * _L, _L)
                fx = jnp.minimum(jnp.maximum(
                    ptsv[sx] * _INV_SCALE + 0.5, 0.0), _CLIP_HI) * (_W - 1.0)
                fy = jnp.minimum(jnp.maximum(
                    ptsv[sy] * _INV_SCALE + 0.5, 0.0), _CLIP_HI) * (_H - 1.0)
                fz = jnp.minimum(jnp.maximum(
                    ptsv[sz] * _INV_SCALE + 0.5, 0.0), _CLIP_HI) * (_D - 1.0)
                x0 = fx.astype(jnp.int32)
                y0 = fy.astype(jnp.int32)
                z0 = fz.astype(jnp.int32)
                wx = fx - x0.astype(jnp.float32)
                wy = fy - y0.astype(jnp.float32)
                wz = fz - z0.astype(jnp.float32)
                ex = 1.0 - wx
                ey = 1.0 - wy
                ez = 1.0 - wz
                base = (z0 * _H + y0) * _W + x0 + tab_off
                for c in range(8):
                    idxs[slot][c][s] = base + _CORNER_OFF[c]
                pzy0 = ez * ey
                pzy1 = ez * wy
                pzy2 = wz * ey
                pzy3 = wz * wy
                w8[0, s] = pzy0 * ex
                w8[1, s] = pzy0 * wx
                w8[2, s] = pzy1 * ex
                w8[3, s] = pzy1 * wx
                w8[4, s] = pzy2 * ex
                w8[5, s] = pzy2 * wx
                w8[6, s] = pzy3 * ex
                w8[7, s] = pzy3 * wx
            for c in range(8):
                pltpu.async_copy(table_hbm.at[idxs[slot][c]], rows[slot][c], gsems[slot])

        def drain_gathers(slot):
            for c in range(8):
                pltpu.make_async_copy(
                    table_hbm.at[idxs[slot][c]], rows[slot][c], gsems[slot]).wait()

        def drain_out(slot):
            pltpu.make_async_copy(
                ovs[slot], out_hbm.at[pl.ds(0, _P)], osems[slot]).wait()

        def accum_store(ci, slot, first):
            n0 = pbase + ci * _P
            w8 = w8s[slot]
            r0, r1, r2, r3, r4, r5, r6, r7 = rows[slot]
            ov = ovs[slot]
            if not first:
                drain_out(slot)

            rr = (r0, r1, r2, r3, r4, r5, r6, r7)

            def grp_body(g, acc_carry):
                p0 = g * _L
                gs = pl.ds(p0, _L)
                wv = [w8[c, gs] for c in range(8)]
                for j in range(_L):
                    p = p0 + j
                    ws = [wv[c][j] for c in range(8)]
                    for kq in range(_C // _L):
                        sl = pl.ds(kq * _L, _L)
                        acc = ws[0] * rr[0][p, sl]
                        for c in range(1, 8):
                            acc = acc + ws[c] * rr[c][p, sl]
                        ov[p, sl] = acc
                return acc_carry

            lax.fori_loop(0, _P // _L, grp_body, 0)
            pltpu.async_copy(ov, out_hbm.at[pl.ds(n0, _P)], osems[slot])

        load_pair_pts(0)
        prep_fire(0, 0, 0)

        def pair_body(q, carry):
            c0 = 2 * q
            prep_fire(c0 + 1, 1, 1)
            drain_gathers(0)

            @pl.when(q > 0)
            def _():
                drain_out(0)
            accum_store(c0, 0, first=True)

            @pl.when(q < _NPAIR - 1)
            def _():
                load_pair_pts(q + 1)
                prep_fire(c0 + 2, 0, 0)
            drain_gathers(1)

            @pl.when(q > 0)
            def _():
                drain_out(1)
            accum_store(c0 + 1, 1, first=True)
            return carry

        lax.fori_loop(0, _NPAIR, pair_body, 0)
        drain_out(0)
        drain_out(1)

    return k(points3, table)


def kernel(points, features_grid):
    B, N, _ = points.shape
    _, C, D, H, W = features_grid.shape
    table = jnp.transpose(features_grid, (0, 2, 3, 4, 1)).reshape(
        B * D * H * W, C)
    npair = B * N // (2 * _P)
    points3 = jnp.transpose(
        jnp.transpose(points, (2, 0, 1)).reshape(3, npair, 2 * _P),
        (1, 0, 2)).reshape(3 * B * N)
    out = _sc_grid_sample(points3, table)
    return out.reshape(B, N, C)
